# baseline (speedup 1.0000x reference)
import functools

import jax
import jax.numpy as jnp
from jax import lax
from jax.experimental import pallas as pl
from jax.experimental.pallas import tpu as pltpu
from jax.experimental.pallas import tpu_sc as plsc

B = 4096
D_IN = 2048
D_OUT = 128
NUM_CLASSES = 100

_NUM_CORES = 1
_NUM_SUBCORES = 16
_NW = _NUM_CORES * _NUM_SUBCORES
_ROWS = B // _NW
_CPAD = 128
_ZROWS = _CPAD // _NUM_SUBCORES
_MM_BLK = 2048
_MM_GRID = B // _MM_BLK
_D_BLK = 4096
_D_GRID = B // _D_BLK


def _mm_body(x_ref, w_ref, b_ref, yf_ref, o_ref, cnt_ref):
    i = pl.program_id(0)
    x = x_ref[...].astype(jnp.bfloat16)
    w = w_ref[...].astype(jnp.bfloat16)
    o_ref[...] = jnp.dot(x, w, preferred_element_type=jnp.float32) + b_ref[...]
    cls = lax.broadcasted_iota(jnp.int32, (NUM_CLASSES, _MM_BLK), 0)
    eq = (yf_ref[...] == cls).astype(jnp.float32)
    partial = jnp.sum(eq, axis=1, keepdims=True)
    prev = jnp.where(i == 0, 0.0, cnt_ref[...])
    cnt_ref[...] = prev + partial


def _embed(images, W, b2d, y_f):
    return pl.pallas_call(
        _mm_body,
        grid=(_MM_GRID,),
        in_specs=[
            pl.BlockSpec((_MM_BLK, D_IN), lambda i: (i, 0)),
            pl.BlockSpec((D_IN, D_OUT), lambda i: (0, 0)),
            pl.BlockSpec((1, D_OUT), lambda i: (0, 0)),
            pl.BlockSpec((1, _MM_BLK), lambda i: (0, i)),
        ],
        out_specs=[
            pl.BlockSpec((_MM_BLK, D_OUT), lambda i: (i, 0)),
            pl.BlockSpec((NUM_CLASSES, 1), lambda i: (0, 0)),
        ],
        out_shape=[
            jax.ShapeDtypeStruct((B, D_OUT), jnp.float32),
            jax.ShapeDtypeStruct((NUM_CLASSES, 1), jnp.float32),
        ],
    )(images, W, b2d, y_f)


def _sc_body(emb_hbm, y_hbm, sums_hbm,
             emb_v, y_v, z_v, sh_sums, sem_y, sem_e):
    cid = lax.axis_index("c")
    sid = lax.axis_index("s")
    wid = sid * _NUM_CORES + cid
    base = wid * _ROWS
    cp_y = pltpu.async_copy(y_hbm.at[pl.ds(base, _ROWS)], y_v, sem_y)
    cp_e = pltpu.async_copy(emb_hbm.at[pl.ds(base, _ROWS)], emb_v, sem_e)

    for r in range(_ZROWS):
        for k in range(D_OUT // 16):
            z_v[r, pl.ds(16 * k, 16)] = jnp.zeros((16,), jnp.float32)
    pltpu.sync_copy(z_v, sh_sums.at[pl.ds(sid * _ZROWS, _ZROWS)])

    cp_y.wait()
    cp_e.wait()
    plsc.subcore_barrier()
    pltpu.sync_copy(emb_v, sh_sums.at[y_v], add=True)
    plsc.subcore_barrier()

    @pl.when(sid == 0)
    def _writeback():
        pltpu.sync_copy(sh_sums.at[pl.ds(0, NUM_CLASSES)], sums_hbm.at[cid])


@functools.lru_cache(maxsize=1)
def _sc_segsum_fn():
    return pl.kernel(
        _sc_body,
        out_type=jax.ShapeDtypeStruct((_NUM_CORES, NUM_CLASSES, D_OUT),
                                      jnp.float32),
        mesh=plsc.VectorSubcoreMesh(core_axis_name="c", subcore_axis_name="s",
                                    num_cores=1),
        scratch_types=[
            pltpu.VMEM((_ROWS, D_OUT), jnp.float32),
            pltpu.VMEM((_ROWS,), jnp.int32),
            pltpu.VMEM((_ZROWS, D_OUT), jnp.float32),
            pltpu.VMEM_SHARED((_CPAD, D_OUT), jnp.float32),
            pltpu.SemaphoreType.DMA,
            pltpu.SemaphoreType.DMA,
        ],
    )


def _sc_segsum(emb, y):
    return _sc_segsum_fn()(emb, y)


def _dist_body(e_ref, s_ref, c_ref, p_ref, ctr_ref, o_ref):
    sums = s_ref[0]
    counts = c_ref[...]
    newp = sums / jnp.maximum(counts, 1.0)
    ctr = ctr_ref[...]
    proto = p_ref[...]
    upd = jnp.where(counts > 0, (ctr * proto + newp) / (ctr + 1.0), proto)
    psq = jnp.sum(upd * upd, axis=1, keepdims=True)
    e = e_ref[...]
    esq_t = jnp.sum(e * e, axis=1)[None, :]
    dots_t = lax.dot_general(upd, e, (((1,), (1,)), ((), ())),
                             preferred_element_type=jnp.float32)
    o_ref[...] = -jnp.sqrt(jnp.maximum(psq + esq_t - 2.0 * dots_t, 0.0))


def _dists_t(emb, sums, cnt, proto, ctr2d):
    return pl.pallas_call(
        _dist_body,
        grid=(_D_GRID,),
        in_specs=[
            pl.BlockSpec((_D_BLK, D_OUT), lambda i: (i, 0)),
            pl.BlockSpec((_NUM_CORES, NUM_CLASSES, D_OUT), lambda i: (0, 0, 0)),
            pl.BlockSpec((NUM_CLASSES, 1), lambda i: (0, 0)),
            pl.BlockSpec((NUM_CLASSES, D_OUT), lambda i: (0, 0)),
            pl.BlockSpec((NUM_CLASSES, 1), lambda i: (0, 0)),
        ],
        out_specs=pl.BlockSpec((NUM_CLASSES, _D_BLK), lambda i: (0, i)),
        out_shape=jax.ShapeDtypeStruct((NUM_CLASSES, B), jnp.float32),
    )(emb, sums, cnt, proto, ctr2d)


def kernel(images, y, W, b, centroid_prototypes, counter):
    y_f = y.reshape(1, B)
    emb, counts = _embed(images, W, b.reshape(1, D_OUT), y_f)
    sums = _sc_segsum(emb, y)
    dists_t = _dists_t(emb, sums, counts, centroid_prototypes,
                       counter.reshape(NUM_CLASSES, 1))
    return dists_t.T, emb


# speedup vs baseline: 1.0103x; 1.0103x over previous
import functools

import jax
import jax.numpy as jnp
from jax import lax
from jax.experimental import pallas as pl
from jax.experimental.pallas import tpu as pltpu
from jax.experimental.pallas import tpu_sc as plsc

B = 4096
D_IN = 2048
D_OUT = 128
NUM_CLASSES = 100

_NUM_CORES = 2
_NUM_SUBCORES = 16
_NW = _NUM_CORES * _NUM_SUBCORES
_ROWS = B // _NW
_CPAD = 128
_ZROWS = _CPAD // _NUM_SUBCORES
_MM_BLK = 2048
_MM_GRID = B // _MM_BLK
_D_BLK = 4096
_D_GRID = B // _D_BLK


def _mm_body(x_ref, w_ref, b_ref, yf_ref, o_ref, cnt_ref):
    i = pl.program_id(0)
    o_ref[...] = (
        jnp.dot(x_ref[...], w_ref[...], preferred_element_type=jnp.float32)
        + b_ref[...]
    )
    cls = lax.broadcasted_iota(jnp.int32, (NUM_CLASSES, _MM_BLK), 0)
    eq = (yf_ref[...] == cls).astype(jnp.float32)
    partial = jnp.sum(eq, axis=1, keepdims=True)
    prev = jnp.where(i == 0, 0.0, cnt_ref[...])
    cnt_ref[...] = prev + partial


def _embed(images, W, b2d, y_f):
    return pl.pallas_call(
        _mm_body,
        grid=(_MM_GRID,),
        in_specs=[
            pl.BlockSpec((_MM_BLK, D_IN), lambda i: (i, 0)),
            pl.BlockSpec((D_IN, D_OUT), lambda i: (0, 0)),
            pl.BlockSpec((1, D_OUT), lambda i: (0, 0)),
            pl.BlockSpec((1, _MM_BLK), lambda i: (0, i)),
        ],
        out_specs=[
            pl.BlockSpec((_MM_BLK, D_OUT), lambda i: (i, 0)),
            pl.BlockSpec((NUM_CLASSES, 1), lambda i: (0, 0)),
        ],
        out_shape=[
            jax.ShapeDtypeStruct((B, D_OUT), jnp.float32),
            jax.ShapeDtypeStruct((NUM_CLASSES, 1), jnp.float32),
        ],
    )(images, W, b2d, y_f)


def _sc_body(emb_hbm, y_hbm, sums_hbm,
             emb_v, y_v, z_v, sh_sums, sem_y, sem_e):
    cid = lax.axis_index("c")
    sid = lax.axis_index("s")
    wid = sid * _NUM_CORES + cid
    base = wid * _ROWS
    cp_y = pltpu.async_copy(y_hbm.at[pl.ds(base, _ROWS)], y_v, sem_y)
    cp_e = pltpu.async_copy(emb_hbm.at[pl.ds(base, _ROWS)], emb_v, sem_e)

    for r in range(_ZROWS):
        for k in range(D_OUT // 16):
            z_v[r, pl.ds(16 * k, 16)] = jnp.zeros((16,), jnp.float32)
    pltpu.sync_copy(z_v, sh_sums.at[pl.ds(sid * _ZROWS, _ZROWS)])

    cp_y.wait()
    cp_e.wait()
    plsc.subcore_barrier()
    pltpu.sync_copy(emb_v, sh_sums.at[y_v], add=True)
    plsc.subcore_barrier()

    @pl.when(sid == 0)
    def _writeback():
        pltpu.sync_copy(sh_sums.at[pl.ds(0, NUM_CLASSES)], sums_hbm.at[cid])


@functools.lru_cache(maxsize=1)
def _sc_segsum_fn():
    return pl.kernel(
        _sc_body,
        out_type=jax.ShapeDtypeStruct((_NUM_CORES, NUM_CLASSES, D_OUT),
                                      jnp.float32),
        mesh=plsc.VectorSubcoreMesh(core_axis_name="c", subcore_axis_name="s"),
        scratch_types=[
            pltpu.VMEM((_ROWS, D_OUT), jnp.float32),
            pltpu.VMEM((_ROWS,), jnp.int32),
            pltpu.VMEM((_ZROWS, D_OUT), jnp.float32),
            pltpu.VMEM_SHARED((_CPAD, D_OUT), jnp.float32),
            pltpu.SemaphoreType.DMA,
            pltpu.SemaphoreType.DMA,
        ],
    )


def _sc_segsum(emb, y):
    return _sc_segsum_fn()(emb, y)


def _dist_body(e_ref, s_ref, c_ref, p_ref, ctr_ref, o_ref):
    sums = s_ref[0] + s_ref[1]
    counts = c_ref[...]
    newp = sums / jnp.maximum(counts, 1.0)
    ctr = ctr_ref[...]
    proto = p_ref[...]
    upd = jnp.where(counts > 0, (ctr * proto + newp) / (ctr + 1.0), proto)
    psq = jnp.sum(upd * upd, axis=1, keepdims=True)
    e = e_ref[...]
    esq_t = jnp.sum(e * e, axis=1)[None, :]
    dots_t = lax.dot_general(upd, e, (((1,), (1,)), ((), ())),
                             preferred_element_type=jnp.float32)
    o_ref[...] = -jnp.sqrt(jnp.maximum(psq + esq_t - 2.0 * dots_t, 0.0))


def _dists_t(emb, sums, cnt, proto, ctr2d):
    return pl.pallas_call(
        _dist_body,
        grid=(_D_GRID,),
        in_specs=[
            pl.BlockSpec((_D_BLK, D_OUT), lambda i: (i, 0)),
            pl.BlockSpec((_NUM_CORES, NUM_CLASSES, D_OUT), lambda i: (0, 0, 0)),
            pl.BlockSpec((NUM_CLASSES, 1), lambda i: (0, 0)),
            pl.BlockSpec((NUM_CLASSES, D_OUT), lambda i: (0, 0)),
            pl.BlockSpec((NUM_CLASSES, 1), lambda i: (0, 0)),
        ],
        out_specs=pl.BlockSpec((NUM_CLASSES, _D_BLK), lambda i: (0, i)),
        out_shape=jax.ShapeDtypeStruct((NUM_CLASSES, B), jnp.float32),
    )(emb, sums, cnt, proto, ctr2d)


def kernel(images, y, W, b, centroid_prototypes, counter):
    y_f = y.reshape(1, B)
    emb, counts = _embed(images, W, b.reshape(1, D_OUT), y_f)
    sums = _sc_segsum(emb, y)
    dists_t = _dists_t(emb, sums, counts, centroid_prototypes,
                       counter.reshape(NUM_CLASSES, 1))
    return dists_t.T, emb
